# Initial kernel scaffold; baseline (speedup 1.0000x reference)
import jax
import jax.numpy as jnp
from jax.experimental import pallas as pl


def kernel(x, emb, gamma, beta, W, b):
    raise NotImplementedError("write your pallas kernel here")


# speedup vs baseline: 6.3750x; 6.3750x over previous
import functools

import jax
import jax.numpy as jnp
from jax import lax
from jax.experimental import pallas as pl
from jax.experimental.pallas import tpu as pltpu
from jax.experimental.pallas import tpu_sc as plsc

VOCAB = 30000
D_EMB = 512
D_OUT = 64
EPS = 1e-5

ROW_BLOCK = 2000
_PAD = 128


def _table_body(emb_ref, gamma_ref, beta_ref, wt_ref, b_ref, out_ref):
    e = emb_ref[...]
    mean = jnp.mean(e, axis=1, keepdims=True)
    c = e - mean
    var = jnp.mean(c * c, axis=1, keepdims=True)
    h = c * lax.rsqrt(var + EPS) * gamma_ref[...] + beta_ref[...]
    z = jnp.dot(h, wt_ref[...], preferred_element_type=jnp.float32) + b_ref[...]
    z = z - jnp.max(z, axis=1, keepdims=True)
    ez = jnp.exp(z)
    out_ref[...] = ez / jnp.sum(ez, axis=1, keepdims=True)


def _make_table(emb, gamma, beta, W, b):
    nblk = VOCAB // ROW_BLOCK
    return pl.pallas_call(
        _table_body,
        grid=(nblk,),
        in_specs=[
            pl.BlockSpec((ROW_BLOCK, D_EMB), lambda i: (i, 0)),
            pl.BlockSpec((1, D_EMB), lambda i: (0, 0)),
            pl.BlockSpec((1, D_EMB), lambda i: (0, 0)),
            pl.BlockSpec((D_EMB, D_OUT), lambda i: (0, 0)),
            pl.BlockSpec((1, D_OUT), lambda i: (0, 0)),
        ],
        out_specs=pl.BlockSpec((ROW_BLOCK, D_OUT), lambda i: (i, 0)),
        out_shape=jax.ShapeDtypeStruct((VOCAB, D_OUT), jnp.float32),
    )(emb, gamma.reshape(1, D_EMB), beta.reshape(1, D_EMB),
      W.T, b.reshape(1, D_OUT))


try:
    _INFO = plsc.get_sparse_core_info()
    _NC, _NS = _INFO.num_cores, _INFO.num_subcores
except ValueError:
    _NC, _NS = 2, 16
_NW = _NC * _NS
_IW = 128


def _gather_body(j_rows, table_hbm, idx_hbm, out_hbm, idx_v, rows_v, sem):
    wid = lax.axis_index("s") * _NC + lax.axis_index("c")
    pltpu.sync_copy(idx_hbm.at[wid], idx_v)
    base = wid * j_rows * _IW

    def step(j, _):
        pltpu.async_copy(table_hbm.at[idx_v.at[j]], rows_v, sem).wait()
        pltpu.sync_copy(rows_v, out_hbm.at[pl.ds(base + j * _IW, _IW)])
        return 0

    lax.fori_loop(0, j_rows, step, 0, unroll=False)


def _gather(table, idx_flat):
    n = idx_flat.shape[0]
    j_rows = n // (_NW * _IW)
    idx3 = idx_flat.reshape(_NW, j_rows, _IW)
    mesh = plsc.VectorSubcoreMesh(core_axis_name="c", subcore_axis_name="s")
    k = pl.kernel(
        functools.partial(_gather_body, j_rows),
        out_type=jax.ShapeDtypeStruct((n, D_OUT), jnp.float32),
        mesh=mesh,
        scratch_types=[
            pltpu.VMEM((j_rows, _IW), jnp.int32),
            pltpu.VMEM((_IW, D_OUT), jnp.float32),
            pltpu.SemaphoreType.DMA,
        ],
        compiler_params=pltpu.CompilerParams(use_tc_tiling_on_sc=False),
    )
    return k(table, idx3)


def kernel(x, emb, gamma, beta, W, b):
    B, L = x.shape
    table = _make_table(emb, gamma, beta, W, b)
    idx_flat = x.reshape(-1).astype(jnp.int32)
    out = _gather(table, idx_flat)
    return out.reshape(B, L, D_OUT)
